# baseline (speedup 1.0000x reference)
import functools
import jax
import jax.numpy as jnp
from jax import lax
from jax.experimental import pallas as pl
from jax.experimental.pallas import tpu as pltpu
from jax.experimental.pallas import tpu_sc as plsc

_SIG_WEIGHT = 30.0
_CLOSE_MIN = 0.05

_W = 224
_H = 224
_NW = 32
_CPR = _W // 16
_BANDS = 56
_B2 = 24


def _lane_shift(x, idx):
    return lax.gather(
        x, idx[:, None],
        dimension_numbers=lax.GatherDimensionNumbers(
            offset_dims=(), collapsed_slice_dims=(0,), start_index_map=(0,)),
        slice_sizes=(1,),
        mode=lax.GatherScatterMode.PROMISE_IN_BOUNDS)


def _sc_body(r_hbm, a_hbm, out_hbm, a_b0, a_b1, r_b0, r_b1, acc_v, sem):
    c = lax.axis_index("c")
    s = lax.axis_index("s")
    wid = s * 2 + c
    two = wid < _B2
    g0 = jnp.where(two, 2 * wid, wid + _B2)
    g1 = jnp.where(two, 2 * wid + 1, wid + _B2)

    copies = []
    for g, av, rv in ((g0, a_b0, r_b0), (g1, a_b1, r_b1)):
        b = g // 28
        rows = pl.ds((g % 28) * 8, 8)
        copies.append(pltpu.async_copy(
            a_hbm.at[b, 0, rows, pl.ds(0, 128)],
            av.at[:, pl.ds(0, 128)], sem))
        copies.append(pltpu.async_copy(
            a_hbm.at[b, 0, rows, pl.ds(128, 96)],
            av.at[:, pl.ds(128, 96)], sem))
        copies.append(pltpu.async_copy(
            r_hbm.at[b, 0, rows, pl.ds(0, 128)],
            rv.at[:, pl.ds(0, 128)], sem))
        copies.append(pltpu.async_copy(
            r_hbm.at[b, 0, rows, pl.ds(128, 96)],
            rv.at[:, pl.ds(128, 96)], sem))
    for cp in copies:
        cp.wait()

    zero = jnp.zeros((16,), jnp.float32)

    def band_sum(av, rv):
        def do_row(i, accs):
            lane = lax.iota(jnp.int32, 16)
            idx_sl = jnp.minimum(lane + 1, 15)
            idx_sr = jnp.maximum(lane - 1, 0)
            idx_lo = lane * 0
            idx_hi = lane * 0 + 15
            m_self0 = lane >= 1
            m_prev0 = lane >= 2
            m_next = lane < 15
            accs = list(accs)
            a_of = [av[i, pl.ds(16 * ch, 16)] for ch in range(_CPR)]
            for ch in range(_CPR):
                a = a_of[ch]
                o = 16 * ch
                r = rv[i, pl.ds(o, 16)]
                if ch == 0:
                    ap = _lane_shift(a, idx_sr)
                elif ch == 8:
                    ap = jnp.where(lane > 0, _lane_shift(a, idx_sr),
                                   _lane_shift(a_of[7], idx_hi))
                else:
                    ap = av[i, pl.ds(o - 1, 16)]
                if ch == 7:
                    an = jnp.where(m_next, _lane_shift(a, idx_sl),
                                   _lane_shift(a_of[8], idx_lo))
                elif ch == _CPR - 1:
                    an = _lane_shift(a, idx_sl)
                else:
                    an = av[i, pl.ds(o + 1, 16)]
                vn = an != 0.0
                vs = a != 0.0
                vp = ap != 0.0
                if ch == 0:
                    vs_m = m_self0 & vs
                    vp_m = m_prev0 & vp
                else:
                    vs_m, vp_m = vs, vp
                vn_m = (m_next & vn) if ch == _CPR - 1 else vn
                m = jnp.where(vn_m, an,
                              jnp.where(vs_m, a, jnp.where(vp_m, ap, a)))
                d0 = r - a
                orig_mse = d0 * d0
                dm = r - m
                alt = dm * dm * dm + _CLOSE_MIN
                loss = jnp.minimum(orig_mse, alt)
                loss = jnp.where(vs, loss * _SIG_WEIGHT, loss)
                accs[ch % 4] = accs[ch % 4] + loss
            return tuple(accs)

        a0, a1, a2, a3 = lax.fori_loop(0, 8, do_row, (zero, zero, zero, zero))
        return (a0 + a1) + (a2 + a3)

    total0 = band_sum(a_b0, r_b0)
    total1 = band_sum(a_b1, r_b1)
    lane = lax.iota(jnp.int32, 16)
    w1 = jnp.where((lane * 0 + wid) < _B2, 1.0, 0.0)
    acc_v[...] = total0 + w1 * total1
    pltpu.sync_copy(acc_v, out_hbm.at[wid])


def kernel(reconstruction, original):
    mesh = plsc.VectorSubcoreMesh(core_axis_name="c", subcore_axis_name="s")
    fn = functools.partial(
        pl.kernel, mesh=mesh,
        out_type=jax.ShapeDtypeStruct((_NW, 16), jnp.float32),
        scratch_types=[
            pltpu.VMEM((8, _W), jnp.float32),
            pltpu.VMEM((8, _W), jnp.float32),
            pltpu.VMEM((8, _W), jnp.float32),
            pltpu.VMEM((8, _W), jnp.float32),
            pltpu.VMEM((16,), jnp.float32),
            pltpu.SemaphoreType.DMA,
        ],
    )(_sc_body)
    partials = fn(reconstruction, original)
    return jnp.sum(partials) / (2 * _H * _W)


# speedup vs baseline: 1.1813x; 1.1813x over previous
import functools
import jax
import jax.numpy as jnp
from jax import lax
from jax.experimental import pallas as pl
from jax.experimental.pallas import tpu as pltpu
from jax.experimental.pallas import tpu_sc as plsc

_SIG_WEIGHT = 30.0
_CLOSE_MIN = 0.05

_W = 224
_H = 224
_NW = 32
_CPR = _W // 16
_BANDS = 56
_B2 = 24


def _lane_shift(x, idx):
    return lax.gather(
        x, idx[:, None],
        dimension_numbers=lax.GatherDimensionNumbers(
            offset_dims=(), collapsed_slice_dims=(0,), start_index_map=(0,)),
        slice_sizes=(1,),
        mode=lax.GatherScatterMode.PROMISE_IN_BOUNDS)


def _sc_body(r_hbm, a_hbm, out_hbm, a_b0, a_b1, r_b0, r_b1, acc_v, sem):
    c = lax.axis_index("c")
    s = lax.axis_index("s")
    wid = s * 2 + c
    two = wid < _B2
    g0 = jnp.where(two, 2 * wid, wid + _B2)
    g1 = jnp.where(two, 2 * wid + 1, wid + _B2)

    copies = []
    for g, av, rv in ((g0, a_b0, r_b0), (g1, a_b1, r_b1)):
        b = g // 28
        rows = pl.ds((g % 28) * 8, 8)
        copies.append(pltpu.async_copy(
            a_hbm.at[b, 0, rows, pl.ds(0, 128)],
            av.at[:, pl.ds(0, 128)], sem))
        copies.append(pltpu.async_copy(
            a_hbm.at[b, 0, rows, pl.ds(128, 96)],
            av.at[:, pl.ds(128, 96)], sem))
        copies.append(pltpu.async_copy(
            r_hbm.at[b, 0, rows, pl.ds(0, 128)],
            rv.at[:, pl.ds(0, 128)], sem))
        copies.append(pltpu.async_copy(
            r_hbm.at[b, 0, rows, pl.ds(128, 96)],
            rv.at[:, pl.ds(128, 96)], sem))
    for cp in copies:
        cp.wait()

    zero = jnp.zeros((16,), jnp.float32)

    def band_sum(av, rv):
        def chunk_loss(col, r, a, an, ap):
            vn_m = (col < _W - 1) & (an != 0.0)
            vs = a != 0.0
            vs_m = (col >= 1) & vs
            vp_m = (col >= 2) & (ap != 0.0)
            m = jnp.where(vn_m, an, jnp.where(vs_m, a, jnp.where(vp_m, ap, a)))
            d0 = r - a
            orig_mse = d0 * d0
            dm = r - m
            alt = dm * dm * dm + _CLOSE_MIN
            loss = jnp.minimum(orig_mse, alt)
            return jnp.where(vs, loss * _SIG_WEIGHT, loss)

        def do_row(i, accs):
            lane = lax.iota(jnp.int32, 16)
            idx_sl = jnp.minimum(lane + 1, 15)
            idx_sr = jnp.maximum(lane - 1, 0)

            def step(ch, carry):
                acc0, acc1, a, hi = carry
                o16 = 16 * ch
                a_next = av[i, pl.ds(o16 + 16, 16)]
                r = rv[i, pl.ds(o16, 16)]
                col = lane + o16
                sl = _lane_shift(a, idx_sl)
                lo = _lane_shift(a_next, lane * 0)
                an = jnp.where(lane < 15, sl, lo)
                ap = jnp.where(lane > 0, _lane_shift(a, idx_sr), hi)
                loss = chunk_loss(col, r, a, an, ap)
                hi_next = _lane_shift(a, lane * 0 + 15)
                return acc1, acc0 + loss, a_next, hi_next

            a0 = av[i, pl.ds(0, 16)]
            acc0, acc1, a13, hi13 = lax.fori_loop(
                0, _CPR - 1, step, (accs[0], accs[1], a0, zero))
            r13 = rv[i, pl.ds(16 * (_CPR - 1), 16)]
            col = lane + 16 * (_CPR - 1)
            an = _lane_shift(a13, idx_sl)
            ap = jnp.where(lane > 0, _lane_shift(a13, idx_sr), hi13)
            loss = chunk_loss(col, r13, a13, an, ap)
            return acc0 + loss, acc1

        acc0, acc1 = lax.fori_loop(0, 8, do_row, (zero, zero))
        return acc0 + acc1

    total0 = band_sum(a_b0, r_b0)
    total1 = band_sum(a_b1, r_b1)
    lane = lax.iota(jnp.int32, 16)
    w1 = jnp.where((lane * 0 + wid) < _B2, 1.0, 0.0)
    acc_v[...] = total0 + w1 * total1
    pltpu.sync_copy(acc_v, out_hbm.at[wid])


def kernel(reconstruction, original):
    mesh = plsc.VectorSubcoreMesh(core_axis_name="c", subcore_axis_name="s")
    fn = functools.partial(
        pl.kernel, mesh=mesh,
        out_type=jax.ShapeDtypeStruct((_NW, 16), jnp.float32),
        scratch_types=[
            pltpu.VMEM((8, _W), jnp.float32),
            pltpu.VMEM((8, _W), jnp.float32),
            pltpu.VMEM((8, _W), jnp.float32),
            pltpu.VMEM((8, _W), jnp.float32),
            pltpu.VMEM((16,), jnp.float32),
            pltpu.SemaphoreType.DMA,
        ],
    )(_sc_body)
    partials = fn(reconstruction, original)
    return jnp.sum(partials) / (2 * _H * _W)


# speedup vs baseline: 1.2446x; 1.0536x over previous
import functools
import jax
import jax.numpy as jnp
from jax import lax
from jax.experimental import pallas as pl
from jax.experimental.pallas import tpu as pltpu
from jax.experimental.pallas import tpu_sc as plsc

_SIG_WEIGHT = 30.0
_CLOSE_MIN = 0.05

_W = 224
_H = 224
_NW = 32
_CPR = _W // 16
_TC_ROW0 = 32


def _lane_shift(x, idx):
    return lax.gather(
        x, idx[:, None],
        dimension_numbers=lax.GatherDimensionNumbers(
            offset_dims=(), collapsed_slice_dims=(0,), start_index_map=(0,)),
        slice_sizes=(1,),
        mode=lax.GatherScatterMode.PROMISE_IN_BOUNDS)


def _sc_body(r_hbm, a_hbm, out_hbm, a_b, r_b, acc_v, sem):
    c = lax.axis_index("c")
    s = lax.axis_index("s")
    wid = s * 2 + c
    b = wid // 28
    rows = pl.ds(lax.rem(wid, 28) * 8, 8)
    copies = [
        pltpu.async_copy(a_hbm.at[b, 0, rows, pl.ds(0, 128)],
                         a_b.at[:, pl.ds(0, 128)], sem),
        pltpu.async_copy(a_hbm.at[b, 0, rows, pl.ds(128, 96)],
                         a_b.at[:, pl.ds(128, 96)], sem),
        pltpu.async_copy(r_hbm.at[b, 0, rows, pl.ds(0, 128)],
                         r_b.at[:, pl.ds(0, 128)], sem),
        pltpu.async_copy(r_hbm.at[b, 0, rows, pl.ds(128, 96)],
                         r_b.at[:, pl.ds(128, 96)], sem),
    ]
    for cp in copies:
        cp.wait()

    zero = jnp.zeros((16,), jnp.float32)

    def chunk_loss(col, r, a, an, ap):
        vn_m = (col < _W - 1) & (an != 0.0)
        vs = a != 0.0
        vs_m = (col >= 1) & vs
        vp_m = (col >= 2) & (ap != 0.0)
        m = jnp.where(vn_m, an, jnp.where(vs_m, a, jnp.where(vp_m, ap, a)))
        d0 = r - a
        orig_mse = d0 * d0
        dm = r - m
        alt = dm * dm * dm + _CLOSE_MIN
        loss = jnp.minimum(orig_mse, alt)
        return jnp.where(vs, loss * _SIG_WEIGHT, loss)

    def do_row(i, accs):
        lane = lax.iota(jnp.int32, 16)
        idx_sl = jnp.minimum(lane + 1, 15)
        idx_sr = jnp.maximum(lane - 1, 0)

        def step(ch, carry):
            acc0, acc1, a, hi = carry
            o16 = 16 * ch
            a_next = a_b[i, pl.ds(o16 + 16, 16)]
            r = r_b[i, pl.ds(o16, 16)]
            col = lane + o16
            sl = _lane_shift(a, idx_sl)
            lo = _lane_shift(a_next, lane * 0)
            an = jnp.where(lane < 15, sl, lo)
            ap = jnp.where(lane > 0, _lane_shift(a, idx_sr), hi)
            loss = chunk_loss(col, r, a, an, ap)
            hi_next = _lane_shift(a, lane * 0 + 15)
            return acc1, acc0 + loss, a_next, hi_next

        a0 = a_b[i, pl.ds(0, 16)]
        acc0, acc1, a13, hi13 = lax.fori_loop(
            0, _CPR - 1, step, (accs[0], accs[1], a0, zero))
        r13 = r_b[i, pl.ds(16 * (_CPR - 1), 16)]
        col = lane + 16 * (_CPR - 1)
        an = _lane_shift(a13, idx_sl)
        ap = jnp.where(lane > 0, _lane_shift(a13, idx_sr), hi13)
        loss = chunk_loss(col, r13, a13, an, ap)
        return acc0 + loss, acc1

    acc0, acc1 = lax.fori_loop(0, 8, do_row, (zero, zero))
    acc_v[...] = acc0 + acc1
    pltpu.sync_copy(acc_v, out_hbm.at[wid])


def _tc_kernel(r_ref, a_ref, o_ref):
    a = a_ref[0, 0]
    r = r_ref[0, 0]
    row = jax.lax.broadcasted_iota(jnp.int32, a.shape, 0)
    col = jax.lax.broadcasted_iota(jnp.int32, a.shape, 1)
    a_next = jnp.concatenate([a[:, 1:], a[:, :1]], axis=1)
    a_prev = jnp.concatenate([a[:, -1:], a[:, :-1]], axis=1)
    valid_next = (col < _W - 1) & (a_next != 0.0)
    valid_self = (col >= 1) & (a != 0.0)
    valid_prev = (col >= 2) & (a_prev != 0.0)
    m = jnp.where(valid_next, a_next,
                  jnp.where(valid_self, a,
                            jnp.where(valid_prev, a_prev, a)))
    d0 = r - a
    orig_mse = d0 * d0
    dm = r - m
    alt = dm * dm * dm + _CLOSE_MIN
    loss = jnp.minimum(orig_mse, alt)
    loss = jnp.where(a != 0.0, loss * _SIG_WEIGHT, loss)
    loss = jnp.where(row >= _TC_ROW0, loss, 0.0)
    o_ref[0, 0] = jnp.sum(loss)


def kernel(reconstruction, original):
    mesh = plsc.VectorSubcoreMesh(core_axis_name="c", subcore_axis_name="s")
    sc_fn = functools.partial(
        pl.kernel, mesh=mesh,
        out_type=jax.ShapeDtypeStruct((_NW, 16), jnp.float32),
        scratch_types=[
            pltpu.VMEM((8, _W), jnp.float32),
            pltpu.VMEM((8, _W), jnp.float32),
            pltpu.VMEM((16,), jnp.float32),
            pltpu.SemaphoreType.DMA,
        ],
    )(_sc_body)
    partials = sc_fn(reconstruction, original)

    img1 = pl.BlockSpec((1, 1, _H, _W), lambda i: (1, 0, 0, 0))
    tc_sum = pl.pallas_call(
        _tc_kernel,
        grid=(1,),
        out_shape=jax.ShapeDtypeStruct((1, 1), jnp.float32),
        in_specs=[img1, img1],
        out_specs=pl.BlockSpec((1, 1), lambda i: (0, 0),
                               memory_space=pltpu.SMEM),
    )(reconstruction, original)

    return (jnp.sum(partials) + tc_sum[0, 0]) / (2 * _H * _W)


# speedup vs baseline: 1.3133x; 1.0552x over previous
import functools
import jax
import jax.numpy as jnp
from jax import lax
from jax.experimental import pallas as pl
from jax.experimental.pallas import tpu as pltpu
from jax.experimental.pallas import tpu_sc as plsc

_SIG_WEIGHT = 30.0
_CLOSE_MIN = 0.05

_W = 224
_H = 224
_NW = 32
_CPR = _W // 16
_RPW = 4
_TC_ROW0 = _NW * _RPW


def _lane_shift(x, idx):
    return lax.gather(
        x, idx[:, None],
        dimension_numbers=lax.GatherDimensionNumbers(
            offset_dims=(), collapsed_slice_dims=(0,), start_index_map=(0,)),
        slice_sizes=(1,),
        mode=lax.GatherScatterMode.PROMISE_IN_BOUNDS)


def _sc_body(r_hbm, a_hbm, out_hbm, a_b, r_b, acc_v, sem):
    c = lax.axis_index("c")
    s = lax.axis_index("s")
    wid = s * 2 + c
    band = wid // 2
    rows = pl.ds(band * 8, 8)
    copies = [
        pltpu.async_copy(a_hbm.at[0, 0, rows, pl.ds(0, 128)],
                         a_b.at[:, pl.ds(0, 128)], sem),
        pltpu.async_copy(a_hbm.at[0, 0, rows, pl.ds(128, 96)],
                         a_b.at[:, pl.ds(128, 96)], sem),
        pltpu.async_copy(r_hbm.at[0, 0, rows, pl.ds(0, 128)],
                         r_b.at[:, pl.ds(0, 128)], sem),
        pltpu.async_copy(r_hbm.at[0, 0, rows, pl.ds(128, 96)],
                         r_b.at[:, pl.ds(128, 96)], sem),
    ]
    for cp in copies:
        cp.wait()

    zero = jnp.zeros((16,), jnp.float32)
    r0 = lax.rem(wid, 2) * _RPW

    def chunk_loss(col, r, a, an, ap):
        vn_m = (col < _W - 1) & (an != 0.0)
        vs = a != 0.0
        vs_m = (col >= 1) & vs
        vp_m = (col >= 2) & (ap != 0.0)
        m = jnp.where(vn_m, an, jnp.where(vs_m, a, jnp.where(vp_m, ap, a)))
        d0 = r - a
        orig_mse = d0 * d0
        dm = r - m
        alt = dm * dm * dm + _CLOSE_MIN
        loss = jnp.minimum(orig_mse, alt)
        return jnp.where(vs, loss * _SIG_WEIGHT, loss)

    def do_row(ri, accs):
        i = r0 + ri
        lane = lax.iota(jnp.int32, 16)
        idx_sl = jnp.minimum(lane + 1, 15)
        idx_sr = jnp.maximum(lane - 1, 0)

        def one_chunk(ch, acc, a, hi):
            o16 = 16 * ch
            a_next = a_b[i, pl.ds(o16 + 16, 16)]
            r = r_b[i, pl.ds(o16, 16)]
            col = lane + o16
            sl = _lane_shift(a, idx_sl)
            lo = _lane_shift(a_next, lane * 0)
            an = jnp.where(lane < 15, sl, lo)
            ap = jnp.where(lane > 0, _lane_shift(a, idx_sr), hi)
            loss = chunk_loss(col, r, a, an, ap)
            hi_next = _lane_shift(a, lane * 0 + 15)
            return acc + loss, a_next, hi_next

        def step(k, carry):
            acc0, acc1, a, hi = carry
            acc0, a, hi = one_chunk(2 * k, acc0, a, hi)
            acc1, a, hi = one_chunk(2 * k + 1, acc1, a, hi)
            return acc0, acc1, a, hi

        a0 = a_b[i, pl.ds(0, 16)]
        acc0, acc1, a12, hi12 = lax.fori_loop(
            0, (_CPR - 2) // 2, step, (accs[0], accs[1], a0, zero))
        acc0, a13, hi13 = one_chunk(_CPR - 2, acc0, a12, hi12)
        r13 = r_b[i, pl.ds(16 * (_CPR - 1), 16)]
        col = lane + 16 * (_CPR - 1)
        an = _lane_shift(a13, idx_sl)
        ap = jnp.where(lane > 0, _lane_shift(a13, idx_sr), hi13)
        loss = chunk_loss(col, r13, a13, an, ap)
        return acc0 + loss, acc1

    acc0, acc1 = lax.fori_loop(0, _RPW, do_row, (zero, zero))
    acc_v[...] = acc0 + acc1
    pltpu.sync_copy(acc_v, out_hbm.at[wid])


def _tc_kernel(r_ref, a_ref, o_ref):
    img = pl.program_id(0)
    a = a_ref[0, 0]
    r = r_ref[0, 0]
    row = jax.lax.broadcasted_iota(jnp.int32, a.shape, 0)
    col = jax.lax.broadcasted_iota(jnp.int32, a.shape, 1)
    a_next = jnp.concatenate([a[:, 1:], a[:, :1]], axis=1)
    a_prev = jnp.concatenate([a[:, -1:], a[:, :-1]], axis=1)
    valid_next = (col < _W - 1) & (a_next != 0.0)
    valid_self = (col >= 1) & (a != 0.0)
    valid_prev = (col >= 2) & (a_prev != 0.0)
    m = jnp.where(valid_next, a_next,
                  jnp.where(valid_self, a,
                            jnp.where(valid_prev, a_prev, a)))
    d0 = r - a
    orig_mse = d0 * d0
    dm = r - m
    alt = dm * dm * dm + _CLOSE_MIN
    loss = jnp.minimum(orig_mse, alt)
    loss = jnp.where(a != 0.0, loss * _SIG_WEIGHT, loss)
    row0 = jnp.where(img == 0, _TC_ROW0, 0)
    loss = jnp.where(row >= row0, loss, 0.0)
    s = jnp.sum(loss)

    @pl.when(img == 0)
    def _():
        o_ref[0, 0] = s

    @pl.when(img == 1)
    def _():
        o_ref[0, 0] = o_ref[0, 0] + s


def kernel(reconstruction, original):
    mesh = plsc.VectorSubcoreMesh(core_axis_name="c", subcore_axis_name="s")
    sc_fn = functools.partial(
        pl.kernel, mesh=mesh,
        out_type=jax.ShapeDtypeStruct((_NW, 16), jnp.float32),
        scratch_types=[
            pltpu.VMEM((8, _W), jnp.float32),
            pltpu.VMEM((8, _W), jnp.float32),
            pltpu.VMEM((16,), jnp.float32),
            pltpu.SemaphoreType.DMA,
        ],
    )(_sc_body)
    partials = sc_fn(reconstruction, original)

    img = pl.BlockSpec((1, 1, _H, _W), lambda i: (i, 0, 0, 0))
    tc_sum = pl.pallas_call(
        _tc_kernel,
        grid=(2,),
        out_shape=jax.ShapeDtypeStruct((1, 1), jnp.float32),
        in_specs=[img, img],
        out_specs=pl.BlockSpec((1, 1), lambda i: (0, 0),
                               memory_space=pltpu.SMEM),
    )(reconstruction, original)

    return (jnp.sum(partials) + tc_sum[0, 0]) / (2 * _H * _W)


# speedup vs baseline: 1.3909x; 1.0591x over previous
import functools
import jax
import jax.numpy as jnp
from jax import lax
from jax.experimental import pallas as pl
from jax.experimental.pallas import tpu as pltpu
from jax.experimental.pallas import tpu_sc as plsc

_SIG_WEIGHT = 30.0
_CLOSE_MIN = 0.05

_W = 224
_H = 224
_NW = 16
_CPR = _W // 16
_RPW = 8
_TC_ROW0 = _NW * _RPW


def _lane_shift(x, idx):
    return lax.gather(
        x, idx[:, None],
        dimension_numbers=lax.GatherDimensionNumbers(
            offset_dims=(), collapsed_slice_dims=(0,), start_index_map=(0,)),
        slice_sizes=(1,),
        mode=lax.GatherScatterMode.PROMISE_IN_BOUNDS)


def _sc_body(r_hbm, a_hbm, out_hbm, a_b, r_b, acc_v, sem):
    wid = lax.axis_index("s")
    rows = pl.ds(wid * 8, 8)
    copies = [
        pltpu.async_copy(a_hbm.at[0, 0, rows, pl.ds(0, 128)],
                         a_b.at[:, pl.ds(0, 128)], sem),
        pltpu.async_copy(a_hbm.at[0, 0, rows, pl.ds(128, 96)],
                         a_b.at[:, pl.ds(128, 96)], sem),
        pltpu.async_copy(r_hbm.at[0, 0, rows, pl.ds(0, 128)],
                         r_b.at[:, pl.ds(0, 128)], sem),
        pltpu.async_copy(r_hbm.at[0, 0, rows, pl.ds(128, 96)],
                         r_b.at[:, pl.ds(128, 96)], sem),
    ]
    for cp in copies:
        cp.wait()

    zero = jnp.zeros((16,), jnp.float32)
    r0 = 0

    def chunk_loss(col, r, a, an, ap):
        vn_m = (col < _W - 1) & (an != 0.0)
        vs = a != 0.0
        vs_m = (col >= 1) & vs
        vp_m = (col >= 2) & (ap != 0.0)
        m = jnp.where(vn_m, an, jnp.where(vs_m, a, jnp.where(vp_m, ap, a)))
        d0 = r - a
        orig_mse = d0 * d0
        dm = r - m
        alt = dm * dm * dm + _CLOSE_MIN
        loss = jnp.minimum(orig_mse, alt)
        return jnp.where(vs, loss * _SIG_WEIGHT, loss)

    def do_row(ri, accs):
        i = r0 + ri
        lane = lax.iota(jnp.int32, 16)
        idx_sl = jnp.minimum(lane + 1, 15)
        idx_sr = jnp.maximum(lane - 1, 0)

        def one_chunk(ch, acc, a, hi):
            o16 = 16 * ch
            a_next = a_b[i, pl.ds(o16 + 16, 16)]
            r = r_b[i, pl.ds(o16, 16)]
            col = lane + o16
            sl = _lane_shift(a, idx_sl)
            lo = _lane_shift(a_next, lane * 0)
            an = jnp.where(lane < 15, sl, lo)
            ap = jnp.where(lane > 0, _lane_shift(a, idx_sr), hi)
            loss = chunk_loss(col, r, a, an, ap)
            hi_next = _lane_shift(a, lane * 0 + 15)
            return acc + loss, a_next, hi_next

        def step(k, carry):
            acc0, acc1, a, hi = carry
            acc0, a, hi = one_chunk(2 * k, acc0, a, hi)
            acc1, a, hi = one_chunk(2 * k + 1, acc1, a, hi)
            return acc0, acc1, a, hi

        a0 = a_b[i, pl.ds(0, 16)]
        acc0, acc1, a12, hi12 = lax.fori_loop(
            0, (_CPR - 2) // 2, step, (accs[0], accs[1], a0, zero))
        acc0, a13, hi13 = one_chunk(_CPR - 2, acc0, a12, hi12)
        r13 = r_b[i, pl.ds(16 * (_CPR - 1), 16)]
        col = lane + 16 * (_CPR - 1)
        an = _lane_shift(a13, idx_sl)
        ap = jnp.where(lane > 0, _lane_shift(a13, idx_sr), hi13)
        loss = chunk_loss(col, r13, a13, an, ap)
        return acc0 + loss, acc1

    acc0, acc1 = lax.fori_loop(0, _RPW, do_row, (zero, zero))
    acc_v[...] = acc0 + acc1
    pltpu.sync_copy(acc_v, out_hbm.at[wid])


def _tc_kernel(r_ref, a_ref, o_ref):
    img = pl.program_id(0)
    a = a_ref[0, 0]
    r = r_ref[0, 0]
    row = jax.lax.broadcasted_iota(jnp.int32, a.shape, 0)
    col = jax.lax.broadcasted_iota(jnp.int32, a.shape, 1)
    a_next = jnp.concatenate([a[:, 1:], a[:, :1]], axis=1)
    a_prev = jnp.concatenate([a[:, -1:], a[:, :-1]], axis=1)
    valid_next = (col < _W - 1) & (a_next != 0.0)
    valid_self = (col >= 1) & (a != 0.0)
    valid_prev = (col >= 2) & (a_prev != 0.0)
    m = jnp.where(valid_next, a_next,
                  jnp.where(valid_self, a,
                            jnp.where(valid_prev, a_prev, a)))
    d0 = r - a
    orig_mse = d0 * d0
    dm = r - m
    alt = dm * dm * dm + _CLOSE_MIN
    loss = jnp.minimum(orig_mse, alt)
    loss = jnp.where(a != 0.0, loss * _SIG_WEIGHT, loss)
    row0 = jnp.where(img == 0, _TC_ROW0, 0)
    loss = jnp.where(row >= row0, loss, 0.0)
    s = jnp.sum(loss)

    @pl.when(img == 0)
    def _():
        o_ref[0, 0] = s

    @pl.when(img == 1)
    def _():
        o_ref[0, 0] = o_ref[0, 0] + s


def kernel(reconstruction, original):
    mesh = plsc.VectorSubcoreMesh(core_axis_name="c", subcore_axis_name="s",
                                  num_cores=1)
    sc_fn = functools.partial(
        pl.kernel, mesh=mesh,
        out_type=jax.ShapeDtypeStruct((_NW, 16), jnp.float32),
        scratch_types=[
            pltpu.VMEM((8, _W), jnp.float32),
            pltpu.VMEM((8, _W), jnp.float32),
            pltpu.VMEM((16,), jnp.float32),
            pltpu.SemaphoreType.DMA,
        ],
    )(_sc_body)
    partials = sc_fn(reconstruction, original)

    img = pl.BlockSpec((1, 1, _H, _W), lambda i: (i, 0, 0, 0))
    tc_sum = pl.pallas_call(
        _tc_kernel,
        grid=(2,),
        out_shape=jax.ShapeDtypeStruct((1, 1), jnp.float32),
        in_specs=[img, img],
        out_specs=pl.BlockSpec((1, 1), lambda i: (0, 0),
                               memory_space=pltpu.SMEM),
    )(reconstruction, original)

    return (jnp.sum(partials) + tc_sum[0, 0]) / (2 * _H * _W)


# speedup vs baseline: 1.4144x; 1.0169x over previous
import functools
import jax
import jax.numpy as jnp
from jax import lax
from jax.experimental import pallas as pl
from jax.experimental.pallas import tpu as pltpu
from jax.experimental.pallas import tpu_sc as plsc

_SIG_WEIGHT = 30.0
_CLOSE_MIN = 0.05

_W = 224
_H = 224
_NW = 16
_CPR = _W // 16
_RPW = 4
_TC_ROW0 = _NW * _RPW


def _lane_shift(x, idx):
    return lax.gather(
        x, idx[:, None],
        dimension_numbers=lax.GatherDimensionNumbers(
            offset_dims=(), collapsed_slice_dims=(0,), start_index_map=(0,)),
        slice_sizes=(1,),
        mode=lax.GatherScatterMode.PROMISE_IN_BOUNDS)


def _sc_body(r_hbm, a_hbm, out_hbm, a_b, r_b, acc_v, sem):
    wid = lax.axis_index("s")
    rows = pl.ds((wid // 2) * 8, 8)
    copies = [
        pltpu.async_copy(a_hbm.at[0, 0, rows, pl.ds(0, 128)],
                         a_b.at[:, pl.ds(0, 128)], sem),
        pltpu.async_copy(a_hbm.at[0, 0, rows, pl.ds(128, 96)],
                         a_b.at[:, pl.ds(128, 96)], sem),
        pltpu.async_copy(r_hbm.at[0, 0, rows, pl.ds(0, 128)],
                         r_b.at[:, pl.ds(0, 128)], sem),
        pltpu.async_copy(r_hbm.at[0, 0, rows, pl.ds(128, 96)],
                         r_b.at[:, pl.ds(128, 96)], sem),
    ]
    for cp in copies:
        cp.wait()

    zero = jnp.zeros((16,), jnp.float32)
    r0 = lax.rem(wid, 2) * _RPW

    def chunk_loss(col, r, a, an, ap):
        vn_m = (col < _W - 1) & (an != 0.0)
        vs = a != 0.0
        vs_m = (col >= 1) & vs
        vp_m = (col >= 2) & (ap != 0.0)
        m = jnp.where(vn_m, an, jnp.where(vs_m, a, jnp.where(vp_m, ap, a)))
        d0 = r - a
        orig_mse = d0 * d0
        dm = r - m
        alt = dm * dm * dm + _CLOSE_MIN
        loss = jnp.minimum(orig_mse, alt)
        return jnp.where(vs, loss * _SIG_WEIGHT, loss)

    def do_row(ri, accs):
        i = r0 + ri
        lane = lax.iota(jnp.int32, 16)
        idx_sl = jnp.minimum(lane + 1, 15)
        idx_sr = jnp.maximum(lane - 1, 0)

        def one_chunk(ch, acc, a, hi):
            o16 = 16 * ch
            a_next = a_b[i, pl.ds(o16 + 16, 16)]
            r = r_b[i, pl.ds(o16, 16)]
            col = lane + o16
            sl = _lane_shift(a, idx_sl)
            lo = _lane_shift(a_next, lane * 0)
            an = jnp.where(lane < 15, sl, lo)
            ap = jnp.where(lane > 0, _lane_shift(a, idx_sr), hi)
            loss = chunk_loss(col, r, a, an, ap)
            hi_next = _lane_shift(a, lane * 0 + 15)
            return acc + loss, a_next, hi_next

        def step(k, carry):
            acc0, acc1, a, hi = carry
            acc0, a, hi = one_chunk(2 * k, acc0, a, hi)
            acc1, a, hi = one_chunk(2 * k + 1, acc1, a, hi)
            return acc0, acc1, a, hi

        a0 = a_b[i, pl.ds(0, 16)]
        acc0, acc1, a12, hi12 = lax.fori_loop(
            0, (_CPR - 2) // 2, step, (accs[0], accs[1], a0, zero))
        acc0, a13, hi13 = one_chunk(_CPR - 2, acc0, a12, hi12)
        r13 = r_b[i, pl.ds(16 * (_CPR - 1), 16)]
        col = lane + 16 * (_CPR - 1)
        an = _lane_shift(a13, idx_sl)
        ap = jnp.where(lane > 0, _lane_shift(a13, idx_sr), hi13)
        loss = chunk_loss(col, r13, a13, an, ap)
        return acc0 + loss, acc1

    acc0, acc1 = lax.fori_loop(0, _RPW, do_row, (zero, zero))
    acc_v[...] = acc0 + acc1
    pltpu.sync_copy(acc_v, out_hbm.at[wid])


def _tc_kernel(r_ref, a_ref, o_ref):
    img = pl.program_id(0)
    a = a_ref[0, 0]
    r = r_ref[0, 0]
    row = jax.lax.broadcasted_iota(jnp.int32, a.shape, 0)
    col = jax.lax.broadcasted_iota(jnp.int32, a.shape, 1)
    a_next = jnp.concatenate([a[:, 1:], a[:, :1]], axis=1)
    a_prev = jnp.concatenate([a[:, -1:], a[:, :-1]], axis=1)
    valid_next = (col < _W - 1) & (a_next != 0.0)
    valid_self = (col >= 1) & (a != 0.0)
    valid_prev = (col >= 2) & (a_prev != 0.0)
    m = jnp.where(valid_next, a_next,
                  jnp.where(valid_self, a,
                            jnp.where(valid_prev, a_prev, a)))
    d0 = r - a
    orig_mse = d0 * d0
    dm = r - m
    alt = dm * dm * dm + _CLOSE_MIN
    loss = jnp.minimum(orig_mse, alt)
    loss = jnp.where(a != 0.0, loss * _SIG_WEIGHT, loss)
    row0 = jnp.where(img == 0, _TC_ROW0, 0)
    loss = jnp.where(row >= row0, loss, 0.0)
    s = jnp.sum(loss)

    @pl.when(img == 0)
    def _():
        o_ref[0, 0] = s

    @pl.when(img == 1)
    def _():
        o_ref[0, 0] = o_ref[0, 0] + s


def kernel(reconstruction, original):
    mesh = plsc.VectorSubcoreMesh(core_axis_name="c", subcore_axis_name="s",
                                  num_cores=1)
    sc_fn = functools.partial(
        pl.kernel, mesh=mesh,
        out_type=jax.ShapeDtypeStruct((_NW, 16), jnp.float32),
        scratch_types=[
            pltpu.VMEM((8, _W), jnp.float32),
            pltpu.VMEM((8, _W), jnp.float32),
            pltpu.VMEM((16,), jnp.float32),
            pltpu.SemaphoreType.DMA,
        ],
    )(_sc_body)
    partials = sc_fn(reconstruction, original)

    img = pl.BlockSpec((1, 1, _H, _W), lambda i: (i, 0, 0, 0))
    tc_sum = pl.pallas_call(
        _tc_kernel,
        grid=(2,),
        out_shape=jax.ShapeDtypeStruct((1, 1), jnp.float32),
        in_specs=[img, img],
        out_specs=pl.BlockSpec((1, 1), lambda i: (0, 0),
                               memory_space=pltpu.SMEM),
    )(reconstruction, original)

    return (jnp.sum(partials) + tc_sum[0, 0]) / (2 * _H * _W)


# speedup vs baseline: 1.4177x; 1.0023x over previous
import functools
import jax
import jax.numpy as jnp
from jax import lax
from jax.experimental import pallas as pl
from jax.experimental.pallas import tpu as pltpu
from jax.experimental.pallas import tpu_sc as plsc

_SIG_WEIGHT = 30.0
_CLOSE_MIN = 0.05

_W = 224
_H = 224
_NW = 16
_CPR = _W // 16
_RPW = 2
_TC_ROW0 = _NW * _RPW


def _lane_shift(x, idx):
    return lax.gather(
        x, idx[:, None],
        dimension_numbers=lax.GatherDimensionNumbers(
            offset_dims=(), collapsed_slice_dims=(0,), start_index_map=(0,)),
        slice_sizes=(1,),
        mode=lax.GatherScatterMode.PROMISE_IN_BOUNDS)


def _sc_body(r_hbm, a_hbm, out_hbm, a_b, r_b, acc_v, sem):
    wid = lax.axis_index("s")
    rows = pl.ds((wid // 4) * 8, 8)
    copies = [
        pltpu.async_copy(a_hbm.at[0, 0, rows, pl.ds(0, 128)],
                         a_b.at[:, pl.ds(0, 128)], sem),
        pltpu.async_copy(a_hbm.at[0, 0, rows, pl.ds(128, 96)],
                         a_b.at[:, pl.ds(128, 96)], sem),
        pltpu.async_copy(r_hbm.at[0, 0, rows, pl.ds(0, 128)],
                         r_b.at[:, pl.ds(0, 128)], sem),
        pltpu.async_copy(r_hbm.at[0, 0, rows, pl.ds(128, 96)],
                         r_b.at[:, pl.ds(128, 96)], sem),
    ]
    for cp in copies:
        cp.wait()

    zero = jnp.zeros((16,), jnp.float32)
    r0 = lax.rem(wid, 4) * _RPW

    def chunk_loss(col, r, a, an, ap):
        vn_m = (col < _W - 1) & (an != 0.0)
        vs = a != 0.0
        vs_m = (col >= 1) & vs
        vp_m = (col >= 2) & (ap != 0.0)
        m = jnp.where(vn_m, an, jnp.where(vs_m, a, jnp.where(vp_m, ap, a)))
        d0 = r - a
        orig_mse = d0 * d0
        dm = r - m
        alt = dm * dm * dm + _CLOSE_MIN
        loss = jnp.minimum(orig_mse, alt)
        return jnp.where(vs, loss * _SIG_WEIGHT, loss)

    def do_row(ri, accs):
        i = r0 + ri
        lane = lax.iota(jnp.int32, 16)
        idx_sl = jnp.minimum(lane + 1, 15)
        idx_sr = jnp.maximum(lane - 1, 0)

        def one_chunk(ch, acc, a, hi):
            o16 = 16 * ch
            a_next = a_b[i, pl.ds(o16 + 16, 16)]
            r = r_b[i, pl.ds(o16, 16)]
            col = lane + o16
            sl = _lane_shift(a, idx_sl)
            lo = _lane_shift(a_next, lane * 0)
            an = jnp.where(lane < 15, sl, lo)
            ap = jnp.where(lane > 0, _lane_shift(a, idx_sr), hi)
            loss = chunk_loss(col, r, a, an, ap)
            hi_next = _lane_shift(a, lane * 0 + 15)
            return acc + loss, a_next, hi_next

        def step(k, carry):
            acc0, acc1, a, hi = carry
            acc0, a, hi = one_chunk(2 * k, acc0, a, hi)
            acc1, a, hi = one_chunk(2 * k + 1, acc1, a, hi)
            return acc0, acc1, a, hi

        a0 = a_b[i, pl.ds(0, 16)]
        acc0, acc1, a12, hi12 = lax.fori_loop(
            0, (_CPR - 2) // 2, step, (accs[0], accs[1], a0, zero))
        acc0, a13, hi13 = one_chunk(_CPR - 2, acc0, a12, hi12)
        r13 = r_b[i, pl.ds(16 * (_CPR - 1), 16)]
        col = lane + 16 * (_CPR - 1)
        an = _lane_shift(a13, idx_sl)
        ap = jnp.where(lane > 0, _lane_shift(a13, idx_sr), hi13)
        loss = chunk_loss(col, r13, a13, an, ap)
        return acc0 + loss, acc1

    acc0, acc1 = lax.fori_loop(0, _RPW, do_row, (zero, zero))
    acc_v[...] = acc0 + acc1
    pltpu.sync_copy(acc_v, out_hbm.at[wid])


def _tc_kernel(r_ref, a_ref, o_ref):
    img = pl.program_id(0)
    a = a_ref[0, 0]
    r = r_ref[0, 0]
    row = jax.lax.broadcasted_iota(jnp.int32, a.shape, 0)
    col = jax.lax.broadcasted_iota(jnp.int32, a.shape, 1)
    a_next = jnp.concatenate([a[:, 1:], a[:, :1]], axis=1)
    a_prev = jnp.concatenate([a[:, -1:], a[:, :-1]], axis=1)
    valid_next = (col < _W - 1) & (a_next != 0.0)
    valid_self = (col >= 1) & (a != 0.0)
    valid_prev = (col >= 2) & (a_prev != 0.0)
    m = jnp.where(valid_next, a_next,
                  jnp.where(valid_self, a,
                            jnp.where(valid_prev, a_prev, a)))
    d0 = r - a
    orig_mse = d0 * d0
    dm = r - m
    alt = dm * dm * dm + _CLOSE_MIN
    loss = jnp.minimum(orig_mse, alt)
    loss = jnp.where(a != 0.0, loss * _SIG_WEIGHT, loss)
    row0 = jnp.where(img == 0, _TC_ROW0, 0)
    loss = jnp.where(row >= row0, loss, 0.0)
    s = jnp.sum(loss)

    @pl.when(img == 0)
    def _():
        o_ref[0, 0] = s

    @pl.when(img == 1)
    def _():
        o_ref[0, 0] = o_ref[0, 0] + s


def kernel(reconstruction, original):
    mesh = plsc.VectorSubcoreMesh(core_axis_name="c", subcore_axis_name="s",
                                  num_cores=1)
    sc_fn = functools.partial(
        pl.kernel, mesh=mesh,
        out_type=jax.ShapeDtypeStruct((_NW, 16), jnp.float32),
        scratch_types=[
            pltpu.VMEM((8, _W), jnp.float32),
            pltpu.VMEM((8, _W), jnp.float32),
            pltpu.VMEM((16,), jnp.float32),
            pltpu.SemaphoreType.DMA,
        ],
    )(_sc_body)
    partials = sc_fn(reconstruction, original)

    img = pl.BlockSpec((1, 1, _H, _W), lambda i: (i, 0, 0, 0))
    tc_sum = pl.pallas_call(
        _tc_kernel,
        grid=(2,),
        out_shape=jax.ShapeDtypeStruct((1, 1), jnp.float32),
        in_specs=[img, img],
        out_specs=pl.BlockSpec((1, 1), lambda i: (0, 0),
                               memory_space=pltpu.SMEM),
    )(reconstruction, original)

    return (jnp.sum(partials) + tc_sum[0, 0]) / (2 * _H * _W)
